# baseline (speedup 1.0000x reference)
import functools

import jax
import jax.numpy as jnp
from jax import lax
from jax.experimental import pallas as pl
from jax.experimental.pallas import tpu as pltpu
from jax.experimental.pallas import tpu_sc as plsc

N_NODES = 10000
N_EDGES = 320000
D = 128
DW = D // 2

CHUNK = 128
STAGE = 21
FAST_CORE = 0
FAST_CHUNKS = 147
SLOW_CHUNKS = 21
TOTAL_CHUNKS = 16 * (FAST_CHUNKS + SLOW_CHUNKS)
E_PAD = TOTAL_CHUNKS * CHUNK
N_PAD = 10112
ROWS_PER_TILE = N_PAD // 16


def _sc_aggregate(src2d, dst2d, x_pack):
    mesh = plsc.VectorSubcoreMesh(core_axis_name="c", subcore_axis_name="s")

    @functools.partial(
        pl.kernel,
        mesh=mesh,
        compiler_params=pltpu.CompilerParams(use_tc_tiling_on_sc=False),
        out_type=jax.ShapeDtypeStruct((2, N_PAD, D), jnp.float32),
        scratch_types=[
            pltpu.VMEM((STAGE, CHUNK), jnp.int32),
            pltpu.VMEM((STAGE, CHUNK), jnp.int32),
            pltpu.VMEM((CHUNK, DW), jnp.int32),
            pltpu.VMEM((CHUNK, DW), jnp.int32),
            pltpu.VMEM((CHUNK, DW), jnp.int32),
            pltpu.VMEM((CHUNK, D), jnp.float32),
            pltpu.VMEM_SHARED((N_PAD, D), jnp.float32),
            pltpu.SemaphoreType.DMA,
            pltpu.SemaphoreType.DMA,
            pltpu.SemaphoreType.DMA,
        ],
    )
    def agg_kernel(src_hbm, dst_hbm, x_hbm, out_hbm,
                   src_v, dst_v, pbuf0, pbuf1, pbuf2, fbuf, agg_sh,
                   sem0, sem1, sem2):
        c = lax.axis_index("c")
        s = lax.axis_index("s")

        def zero_body(i, carry):
            fbuf[i // 8, pl.ds((i % 8) * 16, 16)] = jnp.zeros((16,), jnp.float32)
            return carry
        lax.fori_loop(0, CHUNK * D // 16, zero_body, 0)
        zbase = s * ROWS_PER_TILE
        nfull = ROWS_PER_TILE // CHUNK
        for k in range(nfull):
            pltpu.sync_copy(fbuf, agg_sh.at[pl.ds(zbase + k * CHUNK, CHUNK)])
        rem = ROWS_PER_TILE - nfull * CHUNK
        pltpu.sync_copy(fbuf.at[pl.ds(0, rem)],
                        agg_sh.at[pl.ds(zbase + nfull * CHUNK, rem)])
        plsc.subcore_barrier()

        pbufs = (pbuf0, pbuf1, pbuf2)
        sems = (sem0, sem1, sem2)
        is_fast = c == FAST_CORE
        cbase = jnp.where(is_fast, s * FAST_CHUNKS,
                          16 * FAST_CHUNKS + s * SLOW_CHUNKS)
        nstages = jnp.where(is_fast, FAST_CHUNKS // STAGE, SLOW_CHUNKS // STAGE)

        def stage_body(stage, carry):
            sb = cbase + stage * STAGE
            pltpu.sync_copy(src_hbm.at[pl.ds(sb, STAGE)], src_v)
            pltpu.sync_copy(dst_hbm.at[pl.ds(sb, STAGE)], dst_v)

            pltpu.async_copy(x_hbm.at[src_v.at[0]], pbufs[0], sems[0])
            pltpu.async_copy(x_hbm.at[src_v.at[1]], pbufs[1], sems[1])

            def chunk_body(jj, carry2):
                for b in range(3):
                    j = jj * 3 + b
                    nxt = j + 2

                    @pl.when(nxt < STAGE)
                    def _():
                        pltpu.async_copy(x_hbm.at[src_v.at[nxt]],
                                         pbufs[(b + 2) % 3], sems[(b + 2) % 3])

                    pltpu.make_async_copy(x_hbm.at[src_v.at[j]],
                                          pbufs[b], sems[b]).wait()

                    def conv_body(r4, carry3):
                        for dr in range(4):
                            r = r4 * 4 + dr
                            for g in range(4):
                                w = pbufs[b][r, pl.ds(g * 16, 16)]
                                lo = lax.bitcast_convert_type(
                                    w << 16, jnp.float32)
                                hi = lax.bitcast_convert_type(
                                    w & jnp.int32(-65536), jnp.float32)
                                fbuf[r, pl.ds(g * 32, 16)] = lo
                                fbuf[r, pl.ds(g * 32 + 16, 16)] = hi
                        return carry3

                    lax.fori_loop(0, CHUNK // 4, conv_body, 0)
                    pltpu.sync_copy(fbuf, agg_sh.at[dst_v.at[j]], add=True)
                return carry2

            lax.fori_loop(0, STAGE // 3, chunk_body, 0)
            return carry

        lax.fori_loop(0, nstages, stage_body, 0)
        plsc.subcore_barrier()

        pltpu.sync_copy(agg_sh.at[pl.ds(zbase, ROWS_PER_TILE)],
                        out_hbm.at[c, pl.ds(zbase, ROWS_PER_TILE)])

    return agg_kernel(src2d, dst2d, x_pack)


def _tc_combine(agg2, x, W_rel, b_rel2, W_root):
    blk = 1000
    grid = N_NODES // blk

    def body(a_ref, x_ref, wrel_ref, wroot_ref, b_ref, o_ref):
        agg = a_ref[0] + a_ref[1]
        dn = (((1,), (1,)), ((), ()))
        o_ref[...] = (
            lax.dot_general(agg, wrel_ref[...], dn,
                            preferred_element_type=jnp.float32)
            + lax.dot_general(x_ref[...], wroot_ref[...], dn,
                              preferred_element_type=jnp.float32)
            + b_ref[...]
        )

    return pl.pallas_call(
        body,
        grid=(grid,),
        in_specs=[
            pl.BlockSpec((2, blk, D), lambda i: (0, i, 0)),
            pl.BlockSpec((blk, D), lambda i: (i, 0)),
            pl.BlockSpec((D, D), lambda i: (0, 0)),
            pl.BlockSpec((D, D), lambda i: (0, 0)),
            pl.BlockSpec((1, D), lambda i: (0, 0)),
        ],
        out_specs=pl.BlockSpec((blk, D), lambda i: (i, 0)),
        out_shape=jax.ShapeDtypeStruct((N_NODES, D), jnp.float32),
    )(agg2, x, W_rel, W_root, b_rel2)


def kernel(x, edge_index, W_rel, b_rel, W_root):
    src = edge_index[0].astype(jnp.int32)
    dst = edge_index[1].astype(jnp.int32)
    pad = E_PAD - N_EDGES
    src2d = jnp.concatenate(
        [src, jnp.zeros((pad,), jnp.int32)]).reshape(-1, CHUNK)
    dst2d = jnp.concatenate(
        [dst, jnp.full((pad,), N_NODES, jnp.int32)]).reshape(-1, CHUNK)
    xb4 = x.astype(jnp.bfloat16).reshape(N_NODES, 4, 2, 16)
    x_pack = jax.lax.bitcast_convert_type(
        xb4.transpose(0, 1, 3, 2), jnp.int32).reshape(N_NODES, DW)
    agg2 = _sc_aggregate(src2d, dst2d, x_pack)
    return _tc_combine(agg2, x, W_rel, b_rel.reshape(1, D), W_root)


# speedup vs baseline: 1.7378x; 1.7378x over previous
import functools

import jax
import jax.numpy as jnp
from jax import lax
from jax.experimental import pallas as pl
from jax.experimental.pallas import tpu as pltpu
from jax.experimental.pallas import tpu_sc as plsc

N_NODES = 10000
N_EDGES = 320000
D = 128
DW = D // 2

CHUNK = 128
STAGE = 16
FAST_CORE = 0
FAST_CHUNKS = 144
SLOW_CHUNKS = 16
TOTAL_CHUNKS = 16 * (FAST_CHUNKS + SLOW_CHUNKS)
E_PAD = TOTAL_CHUNKS * CHUNK
N_PAD = 10112
ROWS_PER_TILE = N_PAD // 16


def _sc_aggregate(src2d, dst2d, x_pack):
    mesh = plsc.VectorSubcoreMesh(core_axis_name="c", subcore_axis_name="s")

    @functools.partial(
        pl.kernel,
        mesh=mesh,
        compiler_params=pltpu.CompilerParams(use_tc_tiling_on_sc=False),
        out_type=jax.ShapeDtypeStruct((2, N_PAD, D), jnp.float32),
        scratch_types=[
            pltpu.VMEM((STAGE, CHUNK), jnp.int32),
            pltpu.VMEM((STAGE, CHUNK), jnp.int32),
            pltpu.VMEM((CHUNK, DW), jnp.int32),
            pltpu.VMEM((CHUNK, DW), jnp.int32),
            pltpu.VMEM((CHUNK, D), jnp.float32),
            pltpu.VMEM_SHARED((N_PAD, D), jnp.float32),
            pltpu.SemaphoreType.DMA,
            pltpu.SemaphoreType.DMA,
        ],
    )
    def agg_kernel(src_hbm, dst_hbm, x_hbm, out_hbm,
                   src_v, dst_v, pbuf0, pbuf1, fbuf, agg_sh, sem0, sem1):
        c = lax.axis_index("c")
        s = lax.axis_index("s")

        def zero_body(i, carry):
            fbuf[i // 8, pl.ds((i % 8) * 16, 16)] = jnp.zeros((16,), jnp.float32)
            return carry
        lax.fori_loop(0, CHUNK * D // 16, zero_body, 0)
        zbase = s * ROWS_PER_TILE
        nfull = ROWS_PER_TILE // CHUNK
        for k in range(nfull):
            pltpu.sync_copy(fbuf, agg_sh.at[pl.ds(zbase + k * CHUNK, CHUNK)])
        rem = ROWS_PER_TILE - nfull * CHUNK
        pltpu.sync_copy(fbuf.at[pl.ds(0, rem)],
                        agg_sh.at[pl.ds(zbase + nfull * CHUNK, rem)])
        plsc.subcore_barrier()

        pbufs = (pbuf0, pbuf1)
        sems = (sem0, sem1)
        is_fast = c == FAST_CORE
        cbase = jnp.where(is_fast, s * FAST_CHUNKS,
                          16 * FAST_CHUNKS + s * SLOW_CHUNKS)
        nstages = jnp.where(is_fast, FAST_CHUNKS // STAGE, SLOW_CHUNKS // STAGE)

        def stage_body(stage, carry):
            sb = cbase + stage * STAGE
            pltpu.sync_copy(src_hbm.at[pl.ds(sb, STAGE)], src_v)
            pltpu.sync_copy(dst_hbm.at[pl.ds(sb, STAGE)], dst_v)

            pltpu.async_copy(x_hbm.at[src_v.at[0]], pbufs[0], sems[0])

            def chunk_body(jj, carry2):
                for b in range(2):
                    j = jj * 2 + b
                    nxt = j + 1

                    @pl.when(nxt < STAGE)
                    def _():
                        pltpu.async_copy(x_hbm.at[src_v.at[nxt]],
                                         pbufs[1 - b], sems[1 - b])

                    pltpu.make_async_copy(x_hbm.at[src_v.at[j]],
                                          pbufs[b], sems[b]).wait()

                    def conv_body(r4, carry3):
                        ws = []
                        for dr in range(4):
                            r = r4 * 4 + dr
                            for g in range(4):
                                ws.append(pbufs[b][r, pl.ds(g * 16, 16)])
                        for dr in range(4):
                            r = r4 * 4 + dr
                            for g in range(4):
                                w = ws[dr * 4 + g]
                                lo = lax.bitcast_convert_type(
                                    w << 16, jnp.float32)
                                hi = lax.bitcast_convert_type(
                                    w & jnp.int32(-65536), jnp.float32)
                                fbuf[r, pl.ds(g * 32, 16)] = lo
                                fbuf[r, pl.ds(g * 32 + 16, 16)] = hi
                        return carry3

                    lax.fori_loop(0, CHUNK // 4, conv_body, 0)
                    pltpu.sync_copy(fbuf, agg_sh.at[dst_v.at[j]], add=True)
                return carry2

            lax.fori_loop(0, STAGE // 2, chunk_body, 0)
            return carry

        lax.fori_loop(0, nstages, stage_body, 0)
        plsc.subcore_barrier()

        pltpu.sync_copy(agg_sh.at[pl.ds(zbase, ROWS_PER_TILE)],
                        out_hbm.at[c, pl.ds(zbase, ROWS_PER_TILE)])

    return agg_kernel(src2d, dst2d, x_pack)


def _tc_combine(agg2, x, W_rel, b_rel2, W_root):
    blk = 1000
    grid = N_NODES // blk

    def body(a_ref, x_ref, wrel_ref, wroot_ref, b_ref, o_ref):
        agg = a_ref[0] + a_ref[1]
        dn = (((1,), (1,)), ((), ()))
        o_ref[...] = (
            lax.dot_general(agg, wrel_ref[...], dn,
                            preferred_element_type=jnp.float32)
            + lax.dot_general(x_ref[...], wroot_ref[...], dn,
                              preferred_element_type=jnp.float32)
            + b_ref[...]
        )

    return pl.pallas_call(
        body,
        grid=(grid,),
        in_specs=[
            pl.BlockSpec((2, blk, D), lambda i: (0, i, 0)),
            pl.BlockSpec((blk, D), lambda i: (i, 0)),
            pl.BlockSpec((D, D), lambda i: (0, 0)),
            pl.BlockSpec((D, D), lambda i: (0, 0)),
            pl.BlockSpec((1, D), lambda i: (0, 0)),
        ],
        out_specs=pl.BlockSpec((blk, D), lambda i: (i, 0)),
        out_shape=jax.ShapeDtypeStruct((N_NODES, D), jnp.float32),
    )(agg2, x, W_rel, W_root, b_rel2)


def kernel(x, edge_index, W_rel, b_rel, W_root):
    src = edge_index[0].astype(jnp.int32)
    dst = edge_index[1].astype(jnp.int32)
    pad = E_PAD - N_EDGES
    src2d = jnp.concatenate(
        [src, jnp.zeros((pad,), jnp.int32)]).reshape(-1, CHUNK)
    dst2d = jnp.concatenate(
        [dst, jnp.full((pad,), N_NODES, jnp.int32)]).reshape(-1, CHUNK)
    xb4 = x.astype(jnp.bfloat16).reshape(N_NODES, 4, 2, 16)
    x_pack = jax.lax.bitcast_convert_type(
        xb4.transpose(0, 1, 3, 2), jnp.int32).reshape(N_NODES, DW)
    agg2 = _sc_aggregate(src2d, dst2d, x_pack)
    return _tc_combine(agg2, x, W_rel, b_rel.reshape(1, D), W_root)


# speedup vs baseline: 1.7634x; 1.0147x over previous
import functools

import jax
import jax.numpy as jnp
from jax import lax
from jax.experimental import pallas as pl
from jax.experimental.pallas import tpu as pltpu
from jax.experimental.pallas import tpu_sc as plsc

N_NODES = 10000
N_EDGES = 320000
D = 128
DW = D // 2

CHUNK = 128
HALF = CHUNK // 2
STAGE = 8
FAST_CORE = 0
FAST_CHUNKS = 152
SLOW_CHUNKS = 8
TOTAL_CHUNKS = 16 * (FAST_CHUNKS + SLOW_CHUNKS)
E_PAD = TOTAL_CHUNKS * CHUNK
N_PAD = 10112
ROWS_PER_TILE = N_PAD // 16


def _sc_aggregate(src2d, dst2d, x_pack):
    mesh = plsc.VectorSubcoreMesh(core_axis_name="c", subcore_axis_name="s")

    @functools.partial(
        pl.kernel,
        mesh=mesh,
        compiler_params=pltpu.CompilerParams(use_tc_tiling_on_sc=False),
        out_type=jax.ShapeDtypeStruct((2, N_PAD, D), jnp.float32),
        scratch_types=[
            pltpu.VMEM((STAGE, CHUNK), jnp.int32),
            pltpu.VMEM((2 * STAGE, HALF), jnp.int32),
            pltpu.VMEM((CHUNK, DW), jnp.int32),
            pltpu.VMEM((CHUNK, DW), jnp.int32),
            pltpu.VMEM((HALF, D), jnp.float32),
            pltpu.VMEM((HALF, D), jnp.float32),
            pltpu.VMEM_SHARED((N_PAD, D), jnp.float32),
            pltpu.SemaphoreType.DMA,
            pltpu.SemaphoreType.DMA,
            pltpu.SemaphoreType.DMA,
            pltpu.SemaphoreType.DMA,
        ],
    )
    def agg_kernel(src_hbm, dst_hbm, x_hbm, out_hbm,
                   src_v, dst_v, pbuf0, pbuf1, fbufA, fbufB, agg_sh,
                   sem0, sem1, ssemA, ssemB):
        c = lax.axis_index("c")
        s = lax.axis_index("s")

        def zero_body(i, carry):
            fbufA[i // 8, pl.ds((i % 8) * 16, 16)] = jnp.zeros((16,), jnp.float32)
            return carry
        lax.fori_loop(0, HALF * D // 16, zero_body, 0)
        zbase = s * ROWS_PER_TILE
        nfull = ROWS_PER_TILE // HALF
        for k in range(nfull):
            pltpu.sync_copy(fbufA, agg_sh.at[pl.ds(zbase + k * HALF, HALF)])
        rem = ROWS_PER_TILE - nfull * HALF
        pltpu.sync_copy(fbufA.at[pl.ds(0, rem)],
                        agg_sh.at[pl.ds(zbase + nfull * HALF, rem)])
        plsc.subcore_barrier()

        pbufs = (pbuf0, pbuf1)
        sems = (sem0, sem1)
        is_fast = c == FAST_CORE
        cbase = jnp.where(is_fast, s * FAST_CHUNKS,
                          16 * FAST_CHUNKS + s * SLOW_CHUNKS)
        nstages = jnp.where(is_fast, FAST_CHUNKS // STAGE, SLOW_CHUNKS // STAGE)

        fbufs = (fbufA, fbufB)
        ssems = (ssemA, ssemB)

        def make_conv(b, h):
            def conv_body(r4, carry3):
                ws = []
                for dr in range(4):
                    r = r4 * 4 + dr
                    for g in range(4):
                        ws.append(pbufs[b][h * HALF + r, pl.ds(g * 16, 16)])
                for dr in range(4):
                    r = r4 * 4 + dr
                    for g in range(4):
                        w = ws[dr * 4 + g]
                        lo = lax.bitcast_convert_type(w << 16, jnp.float32)
                        hi = lax.bitcast_convert_type(
                            w & jnp.int32(-65536), jnp.float32)
                        fbufs[h][r, pl.ds(g * 32, 16)] = lo
                        fbufs[h][r, pl.ds(g * 32 + 16, 16)] = hi
                return carry3
            return conv_body

        def stage_body(stage, carry):
            @pl.when(stage > 0)
            def _():
                for h in range(2):
                    pltpu.make_async_copy(fbufs[h], agg_sh.at[dst_v.at[h]],
                                          ssems[h]).wait()

            sb = cbase + stage * STAGE
            pltpu.sync_copy(src_hbm.at[pl.ds(sb, STAGE)], src_v)
            pltpu.sync_copy(dst_hbm.at[pl.ds(2 * sb, 2 * STAGE)], dst_v)

            pltpu.async_copy(x_hbm.at[src_v.at[0]], pbufs[0], sems[0])

            def chunk_body(jj, carry2):
                for b in range(2):
                    j = jj * 2 + b
                    nxt = j + 1

                    @pl.when(nxt < STAGE)
                    def _():
                        pltpu.async_copy(x_hbm.at[src_v.at[nxt]],
                                         pbufs[1 - b], sems[1 - b])

                    pltpu.make_async_copy(x_hbm.at[src_v.at[j]],
                                          pbufs[b], sems[b]).wait()

                    for h in range(2):
                        @pl.when(j > 0)
                        def _():
                            pltpu.make_async_copy(
                                fbufs[h], agg_sh.at[dst_v.at[2 * j + h]],
                                ssems[h]).wait()
                        lax.fori_loop(0, HALF // 4, make_conv(b, h), 0)
                        pltpu.async_copy(fbufs[h],
                                         agg_sh.at[dst_v.at[2 * j + h]],
                                         ssems[h], add=True)
                return carry2

            lax.fori_loop(0, STAGE // 2, chunk_body, 0)
            return carry

        lax.fori_loop(0, nstages, stage_body, 0)
        for h in range(2):
            pltpu.make_async_copy(fbufs[h], agg_sh.at[dst_v.at[h]],
                                  ssems[h]).wait()
        plsc.subcore_barrier()

        pltpu.sync_copy(agg_sh.at[pl.ds(zbase, ROWS_PER_TILE)],
                        out_hbm.at[c, pl.ds(zbase, ROWS_PER_TILE)])

    return agg_kernel(src2d, dst2d, x_pack)


def _tc_combine(agg2, x, W_rel, b_rel2, W_root):
    blk = 1000
    grid = N_NODES // blk

    def body(a_ref, x_ref, wrel_ref, wroot_ref, b_ref, o_ref):
        agg = a_ref[0] + a_ref[1]
        dn = (((1,), (1,)), ((), ()))
        o_ref[...] = (
            lax.dot_general(agg, wrel_ref[...], dn,
                            preferred_element_type=jnp.float32)
            + lax.dot_general(x_ref[...], wroot_ref[...], dn,
                              preferred_element_type=jnp.float32)
            + b_ref[...]
        )

    return pl.pallas_call(
        body,
        grid=(grid,),
        in_specs=[
            pl.BlockSpec((2, blk, D), lambda i: (0, i, 0)),
            pl.BlockSpec((blk, D), lambda i: (i, 0)),
            pl.BlockSpec((D, D), lambda i: (0, 0)),
            pl.BlockSpec((D, D), lambda i: (0, 0)),
            pl.BlockSpec((1, D), lambda i: (0, 0)),
        ],
        out_specs=pl.BlockSpec((blk, D), lambda i: (i, 0)),
        out_shape=jax.ShapeDtypeStruct((N_NODES, D), jnp.float32),
    )(agg2, x, W_rel, W_root, b_rel2)


def kernel(x, edge_index, W_rel, b_rel, W_root):
    src = edge_index[0].astype(jnp.int32)
    dst = edge_index[1].astype(jnp.int32)
    pad = E_PAD - N_EDGES
    src2d = jnp.concatenate(
        [src, jnp.zeros((pad,), jnp.int32)]).reshape(-1, CHUNK)
    dst2d = jnp.concatenate(
        [dst, jnp.full((pad,), N_NODES, jnp.int32)]).reshape(-1, HALF)
    xb4 = x.astype(jnp.bfloat16).reshape(N_NODES, 4, 2, 16)
    x_pack = jax.lax.bitcast_convert_type(
        xb4.transpose(0, 1, 3, 2), jnp.int32).reshape(N_NODES, DW)
    agg2 = _sc_aggregate(src2d, dst2d, x_pack)
    return _tc_combine(agg2, x, W_rel, b_rel.reshape(1, D), W_root)
